# baseline (speedup 1.0000x reference)
import functools

import jax
import jax.numpy as jnp
from jax import lax
from jax.experimental import pallas as pl
from jax.experimental.pallas import tpu as pltpu
from jax.experimental.pallas import tpu_sc as plsc

R = 128
VMIN = -3.0
VMAX = 3.0

NC, NS, L = 2, 16, 16
NW = NC * NS
C = 1024

_OFFS = [dx * R * R + dy * R for dx in (0, 1) for dy in (0, 1)]


def _tile_body(pos_hbm, tab_hbm, out_hbm, pos_v, idx_v, rows_v, f_v, out_v, sem,
               *, bpw):
    wid = lax.axis_index("s") * NC + lax.axis_index("c")
    wbase = wid * bpw
    nchunk = bpw // C
    lanes = lax.iota(jnp.int32, L)

    def chunk_step(ci, _):
        base = wbase + ci * C
        pltpu.sync_copy(pos_hbm.at[pl.ds(base, C)], pos_v)

        def prep_group(g, _):
            o = g * L
            rows = o + lanes
            col0 = jnp.zeros((L,), jnp.int32)
            x = plsc.load_gather(pos_v, [rows, col0])
            y = plsc.load_gather(pos_v, [rows, col0 + 1])
            z = plsc.load_gather(pos_v, [rows, col0 + 2])

            def prep(p):
                t = (p - VMIN) / (VMAX - VMIN) * (R - 1)
                t = jnp.clip(t, 0.0, R - 1 - 1e-6)
                i0 = t.astype(jnp.int32)
                f = t - i0.astype(jnp.float32)
                return i0, f

            x0, fx = prep(x)
            y0, fy = prep(y)
            z0, fz = prep(z)
            cbase = (x0 * R + y0) * R + z0
            for k, off in enumerate(_OFFS):
                idx_v[k, pl.ds(o, L)] = cbase + off
            f_v[0, pl.ds(o, L)] = fx
            f_v[1, pl.ds(o, L)] = fy
            f_v[2, pl.ds(o, L)] = fz
            return 0

        lax.fori_loop(0, C // L, prep_group, 0)

        copies = [pltpu.async_copy(tab_hbm.at[idx_v.at[k]], rows_v.at[k], sem)
                  for k in range(4)]
        for cp in copies:
            cp.wait()

        def blend_group(g, _):
            o = g * L
            rows = o + lanes
            fx = f_v[0, pl.ds(o, L)]
            fy = f_v[1, pl.ds(o, L)]
            fz = f_v[2, pl.ds(o, L)]
            for c in range(3):
                colc = jnp.full((L,), c, jnp.int32)

                def corner(k, dz):
                    return plsc.load_gather(
                        rows_v, [jnp.full((L,), k, jnp.int32), rows,
                                 colc + 3 * dz])

                c00 = corner(0, 0) * (1 - fz) + corner(0, 1) * fz
                c01 = corner(1, 0) * (1 - fz) + corner(1, 1) * fz
                c10 = corner(2, 0) * (1 - fz) + corner(2, 1) * fz
                c11 = corner(3, 0) * (1 - fz) + corner(3, 1) * fz
                c0 = c00 * (1 - fy) + c01 * fy
                c1 = c10 * (1 - fy) + c11 * fy
                rgb = c0 * (1 - fx) + c1 * fx
                plsc.store_scatter(out_v, [rows, colc], rgb)
            return 0

        lax.fori_loop(0, C // L, blend_group, 0)

        pltpu.sync_copy(out_v, out_hbm.at[pl.ds(base, C)])
        return 0

    lax.fori_loop(0, nchunk, chunk_step, 0)


def kernel(position, grid):
    b = position.shape[0]
    assert b % (NW * C) == 0
    bpw = b // NW
    v = R * R * R
    flat = grid.reshape(v, 3)
    nxt = jnp.concatenate([flat[1:], flat[:1]], axis=0)
    table = jnp.concatenate(
        [flat, nxt, jnp.zeros((v, 2), jnp.float32)], axis=-1)
    mesh = plsc.VectorSubcoreMesh(core_axis_name="c", subcore_axis_name="s",
                                  num_cores=NC, num_subcores=NS)
    run = pl.kernel(
        functools.partial(_tile_body, bpw=bpw),
        out_type=jax.ShapeDtypeStruct((b, 3), jnp.float32),
        mesh=mesh,
        scratch_types=[
            pltpu.VMEM((C, 3), jnp.float32),
            pltpu.VMEM((4, C), jnp.int32),
            pltpu.VMEM((4, C, 8), jnp.float32),
            pltpu.VMEM((3, C), jnp.float32),
            pltpu.VMEM((C, 3), jnp.float32),
            pltpu.SemaphoreType.DMA,
        ],
        compiler_params=pltpu.CompilerParams(needs_layout_passes=False,
                                             use_tc_tiling_on_sc=False),
    )
    return run(position, table)


# speedup vs baseline: 6.7659x; 6.7659x over previous
import functools

import jax
import jax.numpy as jnp
from jax import lax
from jax.experimental import pallas as pl
from jax.experimental.pallas import tpu as pltpu
from jax.experimental.pallas import tpu_sc as plsc

R = 128
VMIN = -3.0
VMAX = 3.0

NC, NS, L = 2, 16, 16
NW = NC * NS
V = R * R * R
TW = 16
CS = 2048
C = 1024

_SC_PARAMS = pltpu.CompilerParams(needs_layout_passes=False,
                                  use_tc_tiling_on_sc=False)


def _wid():
    return lax.axis_index("s") * NC + lax.axis_index("c")


_SLAB = CS + 128


def _build_body(src_hbm, tab_hbm, src_v, dst_v):
    rpw = V // NW
    base = _wid() * rpw
    lanes = lax.iota(jnp.int32, L)
    n3 = R * R * R * 3

    def chunk_step(ci, _):
        a = base + ci * CS
        x = a // (R * R)
        yb = (a - x * R * R) // R
        maxyd = jnp.minimum(16, (R - 1) - yb)
        deltas = []
        for c in range(3):
            a3 = x * (R * R * 3) + c * (R * R) + yb * R
            start = jnp.minimum(a3, n3 - _SLAB)
            deltas.append(a3 - start)
            pltpu.sync_copy(src_hbm.at[pl.ds(start, _SLAB)], src_v.at[c])

        def group(g, _):
            rows = a + g * L + lanes
            yl = g // 8
            zb = (g % 8) * L
            z1c = jnp.minimum(zb + lanes + 1, R - 1)
            for c in range(3):
                colc = jnp.full((L,), c, jnp.int32)
                cc = jnp.full((L,), c, jnp.int32)
                for dy in (0, 1):
                    off = deltas[c] + jnp.minimum(yl + dy, maxyd) * R
                    v0 = src_v[c, pl.ds(off + zb, L)]
                    v1 = plsc.load_gather(src_v, [cc, off + z1c])
                    plsc.store_scatter(dst_v, [rows - a, colc + 6 * dy], v0)
                    plsc.store_scatter(dst_v, [rows - a, colc + 6 * dy + 3],
                                       v1)
            return 0

        lax.fori_loop(0, CS // L, group, 0)
        pltpu.sync_copy(dst_v, tab_hbm.at[pl.ds(a, CS)])
        return 0

    lax.fori_loop(0, rpw // CS, chunk_step, 0)


def _sample_body(idx_hbm, fx_hbm, fy_hbm, fz_hbm, tab_hbm,
                 r_hbm, g_hbm, b_hbm,
                 idx_v, f_v, rows_v, out_v, sems, *, bpw):
    wbase = _wid() * bpw
    nchunk = bpw // C
    lanes = lax.iota(jnp.int32, L)

    def stage_fire(ci, sl):
        base = wbase + ci * C
        pltpu.sync_copy(idx_hbm.at[pl.ds(base, C)], idx_v.at[sl, 0])
        pltpu.sync_copy(fx_hbm.at[pl.ds(base, C)], f_v.at[sl, 0])
        pltpu.sync_copy(fy_hbm.at[pl.ds(base, C)], f_v.at[sl, 1])
        pltpu.sync_copy(fz_hbm.at[pl.ds(base, C)], f_v.at[sl, 2])

        def shift_group(g, _):
            o = g * L
            idx_v[sl, 1, pl.ds(o, L)] = idx_v[sl, 0, pl.ds(o, L)] + R * R
            return 0

        lax.fori_loop(0, C // L, shift_group, 0)
        for k in range(2):
            pltpu.async_copy(tab_hbm.at[idx_v.at[sl, k]], rows_v.at[sl, k],
                             sems.at[sl])

    def wait_gathers(sl):
        for k in range(2):
            pltpu.make_async_copy(tab_hbm.at[idx_v.at[sl, k]],
                                  rows_v.at[sl, k], sems.at[sl]).wait()

    def blend_write(ci, sl):
        base = wbase + ci * C
        slv = jnp.full((L,), sl, jnp.int32)

        def blend_group(g, _):
            o = g * L
            rows = o + lanes
            fx = f_v[sl, 0, pl.ds(o, L)]
            fy = f_v[sl, 1, pl.ds(o, L)]
            fz = f_v[sl, 2, pl.ds(o, L)]
            gz = 1 - fz
            gy = 1 - fy
            gx = 1 - fx
            for c in range(3):
                colc = jnp.full((L,), c, jnp.int32)

                def corner(k, dy, dz):
                    return plsc.load_gather(
                        rows_v, [slv, jnp.full((L,), k, jnp.int32), rows,
                                 colc + (6 * dy + 3 * dz)])

                c00 = corner(0, 0, 0) * gz + corner(0, 0, 1) * fz
                c01 = corner(0, 1, 0) * gz + corner(0, 1, 1) * fz
                c10 = corner(1, 0, 0) * gz + corner(1, 0, 1) * fz
                c11 = corner(1, 1, 0) * gz + corner(1, 1, 1) * fz
                c0 = c00 * gy + c01 * fy
                c1 = c10 * gy + c11 * fy
                out_v[c, pl.ds(o, L)] = c0 * gx + c1 * fx
            return 0

        lax.fori_loop(0, C // L, blend_group, 0)
        pltpu.sync_copy(out_v.at[0], r_hbm.at[pl.ds(base, C)])
        pltpu.sync_copy(out_v.at[1], g_hbm.at[pl.ds(base, C)])
        pltpu.sync_copy(out_v.at[2], b_hbm.at[pl.ds(base, C)])

    stage_fire(0, 0)

    def pipe_step(ci, _):
        sl = lax.rem(ci, 2)
        nsl = lax.rem(ci + 1, 2)

        @pl.when(ci + 1 < nchunk)
        def _():
            stage_fire(ci + 1, nsl)

        wait_gathers(sl)
        blend_write(ci, sl)
        return 0

    lax.fori_loop(0, nchunk, pipe_step, 0)


def kernel(position, grid):
    b = position.shape[0]
    assert b % (NW * C) == 0
    bpw = b // NW
    mesh = plsc.VectorSubcoreMesh(core_axis_name="c", subcore_axis_name="s",
                                  num_cores=NC, num_subcores=NS)

    t = (position - VMIN) / (VMAX - VMIN) * (R - 1)
    t = jnp.clip(t, 0.0, R - 1 - 1e-6)
    i0 = jnp.floor(t).astype(jnp.int32)
    f = t - i0.astype(jnp.float32)
    cbase = (i0[:, 0] * R + i0[:, 1]) * R + i0[:, 2]
    fx, fy, fz = f[:, 0], f[:, 1], f[:, 2]

    gwords = jnp.transpose(grid, (0, 3, 1, 2)).reshape(-1)

    table = pl.kernel(
        _build_body,
        out_type=jax.ShapeDtypeStruct((V, TW), jnp.float32),
        mesh=mesh,
        scratch_types=[
            pltpu.VMEM((3, _SLAB), jnp.float32),
            pltpu.VMEM((CS, TW), jnp.float32),
        ],
        compiler_params=_SC_PARAMS,
    )(gwords)

    rgb = pl.kernel(
        functools.partial(_sample_body, bpw=bpw),
        out_type=[jax.ShapeDtypeStruct((b,), jnp.float32)] * 3,
        mesh=mesh,
        scratch_types=[
            pltpu.VMEM((2, 2, C), jnp.int32),
            pltpu.VMEM((2, 3, C), jnp.float32),
            pltpu.VMEM((2, 2, C, TW), jnp.float32),
            pltpu.VMEM((3, C), jnp.float32),
            pltpu.SemaphoreType.DMA((2,)),
        ],
        compiler_params=_SC_PARAMS,
    )(cbase, fx, fy, fz, table)

    return jnp.stack(rgb, axis=1)


# speedup vs baseline: 7.0293x; 1.0389x over previous
import functools

import jax
import jax.numpy as jnp
from jax import lax
from jax.experimental import pallas as pl
from jax.experimental.pallas import tpu as pltpu
from jax.experimental.pallas import tpu_sc as plsc

R = 128
VMIN = -3.0
VMAX = 3.0

NC, NS, L = 2, 16, 16
NW = NC * NS
V = R * R * R
TW = 16
CS = 2048
C = 2048

_SC_PARAMS = pltpu.CompilerParams(needs_layout_passes=False,
                                  use_tc_tiling_on_sc=False)


def _wid():
    return lax.axis_index("s") * NC + lax.axis_index("c")


_SLAB = CS + 128


def _build_body(src_hbm, tab_hbm, src_v, dst_v):
    rpw = V // NW
    base = _wid() * rpw
    lanes = lax.iota(jnp.int32, L)
    n3 = R * R * R * 3

    def chunk_step(ci, _):
        a = base + ci * CS
        x = a // (R * R)
        yb = (a - x * R * R) // R
        maxyd = jnp.minimum(16, (R - 1) - yb)
        deltas = []
        for dx in (0, 1):
            for c in range(3):
                s = dx * 3 + c
                a3 = (x + dx) * (R * R * 3) + c * (R * R) + yb * R
                start = pl.multiple_of(jnp.clip(a3, 0, n3 - _SLAB), 128)
                deltas.append(jnp.clip(a3 - start, 0, 128))
                pltpu.sync_copy(src_hbm.at[pl.ds(start, _SLAB)], src_v.at[s])

        def group(g, _):
            rows = g * L + lanes
            yl = g // 8
            zb = (g % 8) * L
            z1c = jnp.minimum(zb + lanes + 1, R - 1)
            for dx in (0, 1):
                for c in range(3):
                    s = dx * 3 + c
                    ss = jnp.full((L,), s, jnp.int32)
                    colw = jnp.full((L,), dx * 6 + c, jnp.int32)
                    for dy in (0, 1):
                        off = deltas[s] + jnp.minimum(yl + dy, maxyd) * R
                        v0 = src_v[s, pl.ds(off + zb, L)]
                        v1 = plsc.load_gather(src_v, [ss, off + z1c])
                        pair = plsc.bitcast(
                            plsc.pack(v0, v1, format=plsc.PackFormat.INTERLEAVED),
                            jnp.float32)
                        plsc.store_scatter(dst_v, [rows, colw + 3 * dy], pair)
            return 0

        lax.fori_loop(0, CS // L, group, 0)
        pltpu.sync_copy(dst_v, tab_hbm.at[pl.ds(a, CS)])
        return 0

    lax.fori_loop(0, rpw // CS, chunk_step, 0)


def _sample_body(idx_hbm, fx_hbm, fy_hbm, fz_hbm, tab_hbm,
                 r_hbm, g_hbm, b_hbm,
                 idx_v, f_v, rows_v, out_v, sems, *, bpw):
    wbase = _wid() * bpw
    nchunk = bpw // C
    lanes = lax.iota(jnp.int32, L)

    def stage_fire(ci, sl):
        base = wbase + ci * C
        pltpu.sync_copy(idx_hbm.at[pl.ds(base, C)], idx_v.at[sl])
        pltpu.sync_copy(fx_hbm.at[pl.ds(base, C)], f_v.at[sl, 0])
        pltpu.sync_copy(fy_hbm.at[pl.ds(base, C)], f_v.at[sl, 1])
        pltpu.sync_copy(fz_hbm.at[pl.ds(base, C)], f_v.at[sl, 2])
        pltpu.async_copy(tab_hbm.at[idx_v.at[sl]], rows_v.at[sl],
                         sems.at[sl])

    def wait_gathers(sl):
        pltpu.make_async_copy(tab_hbm.at[idx_v.at[sl]],
                              rows_v.at[sl], sems.at[sl]).wait()

    def blend_write(ci, sl):
        base = wbase + ci * C
        slv = jnp.full((L,), sl, jnp.int32)

        def blend_group(g, _):
            o = g * L
            rows = o + lanes
            fx = f_v[sl, 0, pl.ds(o, L)]
            fy = f_v[sl, 1, pl.ds(o, L)]
            fz = f_v[sl, 2, pl.ds(o, L)]
            gz = 1 - fz
            gy = 1 - fy
            gx = 1 - fx
            for c in range(3):

                def zlerp(dx, dy):
                    word = plsc.load_gather(
                        rows_v, [slv, rows,
                                 jnp.full((L,), dx * 6 + dy * 3 + c,
                                          jnp.int32)])
                    z0, z1 = plsc.unpack(
                        plsc.bitcast(word, jnp.bfloat16),
                        format=plsc.PackFormat.INTERLEAVED)
                    return z0 * gz + z1 * fz

                c0 = zlerp(0, 0) * gy + zlerp(0, 1) * fy
                c1 = zlerp(1, 0) * gy + zlerp(1, 1) * fy
                out_v[c, pl.ds(o, L)] = c0 * gx + c1 * fx
            return 0

        lax.fori_loop(0, C // L, blend_group, 0)
        pltpu.sync_copy(out_v.at[0], r_hbm.at[pl.ds(base, C)])
        pltpu.sync_copy(out_v.at[1], g_hbm.at[pl.ds(base, C)])
        pltpu.sync_copy(out_v.at[2], b_hbm.at[pl.ds(base, C)])

    stage_fire(0, 0)

    def pipe_step(ci, _):
        sl = lax.rem(ci, 2)
        nsl = lax.rem(ci + 1, 2)

        @pl.when(ci + 1 < nchunk)
        def _():
            stage_fire(ci + 1, nsl)

        wait_gathers(sl)
        blend_write(ci, sl)
        return 0

    lax.fori_loop(0, nchunk, pipe_step, 0)


def kernel(position, grid):
    b = position.shape[0]
    assert b % (NW * C) == 0
    bpw = b // NW
    mesh = plsc.VectorSubcoreMesh(core_axis_name="c", subcore_axis_name="s",
                                  num_cores=NC, num_subcores=NS)

    t = (position - VMIN) / (VMAX - VMIN) * (R - 1)
    t = jnp.clip(t, 0.0, R - 1 - 1e-6)
    i0 = jnp.floor(t).astype(jnp.int32)
    f = t - i0.astype(jnp.float32)
    cbase = (i0[:, 0] * R + i0[:, 1]) * R + i0[:, 2]
    fx, fy, fz = f[:, 0], f[:, 1], f[:, 2]

    gwords = jnp.transpose(grid, (0, 3, 1, 2)).reshape(-1)

    table = pl.kernel(
        _build_body,
        out_type=jax.ShapeDtypeStruct((V, TW), jnp.float32),
        mesh=mesh,
        scratch_types=[
            pltpu.VMEM((6, _SLAB), jnp.float32),
            pltpu.VMEM((CS, TW), jnp.float32),
        ],
        compiler_params=_SC_PARAMS,
    )(gwords)

    rgb = pl.kernel(
        functools.partial(_sample_body, bpw=bpw),
        out_type=[jax.ShapeDtypeStruct((b,), jnp.float32)] * 3,
        mesh=mesh,
        scratch_types=[
            pltpu.VMEM((2, C), jnp.int32),
            pltpu.VMEM((2, 3, C), jnp.float32),
            pltpu.VMEM((2, C, TW), jnp.float32),
            pltpu.VMEM((3, C), jnp.float32),
            pltpu.SemaphoreType.DMA((2,)),
        ],
        compiler_params=_SC_PARAMS,
    )(cbase, fx, fy, fz, table)

    return jnp.stack(rgb, axis=1)


# speedup vs baseline: 8.9016x; 1.2664x over previous
import functools

import jax
import jax.numpy as jnp
from jax import lax
from jax.experimental import pallas as pl
from jax.experimental.pallas import tpu as pltpu
from jax.experimental.pallas import tpu_sc as plsc

R = 128
VMIN = -3.0
VMAX = 3.0

NC, NS, L = 2, 16, 16
NW = NC * NS
V = R * R * R
TW = 16
CS = 2048
C = 2048

_SC_PARAMS = pltpu.CompilerParams(needs_layout_passes=False,
                                  use_tc_tiling_on_sc=False)


def _wid():
    return lax.axis_index("s") * NC + lax.axis_index("c")


_SLAB = CS + 136


def _build_body(src_hbm, tab_hbm, src_v, dst_v, sin, sout):
    rpw = V // NW
    base = _wid() * rpw
    nchunk = rpw // CS
    lanes = lax.iota(jnp.int32, L)

    def slab_start(a, dx, c):
        x = a // (R * R)
        yb = (a - x * R * R) // R
        xs = jnp.minimum(x + dx, R - 1)
        return pl.multiple_of(xs * (R * R * 3) + c * (R * R) + yb * R, 128)

    def stage(ci, sl):
        a = base + ci * CS
        for dx in (0, 1):
            for c in range(3):
                pltpu.async_copy(
                    src_hbm.at[pl.ds(slab_start(a, dx, c), _SLAB)],
                    src_v.at[sl, dx * 3 + c], sin.at[sl])

    def wait_stage(ci, sl):
        a = base + ci * CS
        for dx in (0, 1):
            for c in range(3):
                pltpu.make_async_copy(
                    src_hbm.at[pl.ds(slab_start(a, dx, c), _SLAB)],
                    src_v.at[sl, dx * 3 + c], sin.at[sl]).wait()

    def compute(ci, sl):
        a = base + ci * CS

        def group(g, _):
            rows = g * L + lanes
            yl = g // 8
            zb = (g % 8) * L
            for dx in (0, 1):
                for c in range(3):
                    s = dx * 3 + c
                    colw = jnp.full((L,), dx * 6 + c, jnp.int32)
                    for dy in (0, 1):
                        off = (yl + dy) * R + zb
                        v0 = src_v[sl, s, pl.ds(off, L)]
                        v1 = src_v[sl, s, pl.ds(off + 1, L)]
                        pair = plsc.bitcast(
                            plsc.pack(v0, v1,
                                      format=plsc.PackFormat.INTERLEAVED),
                            jnp.float32)
                        plsc.store_scatter(dst_v, [jnp.full((L,), sl, jnp.int32),
                                                   rows, colw + 3 * dy], pair)
            return 0

        lax.fori_loop(0, CS // L, group, 0)
        pltpu.async_copy(dst_v.at[sl], tab_hbm.at[pl.ds(a, CS)], sout.at[sl])

    def wait_out(ci, sl):
        a = base + ci * CS
        pltpu.make_async_copy(dst_v.at[sl], tab_hbm.at[pl.ds(a, CS)],
                              sout.at[sl]).wait()

    stage(0, 0)

    def pipe_step(ci, _):
        sl = lax.rem(ci, 2)
        nsl = lax.rem(ci + 1, 2)

        @pl.when(ci + 1 < nchunk)
        def _():
            stage(ci + 1, nsl)

        wait_stage(ci, sl)

        @pl.when(ci >= 2)
        def _():
            wait_out(ci - 2, sl)

        compute(ci, sl)
        return 0

    lax.fori_loop(0, nchunk, pipe_step, 0)
    wait_out(nchunk - 2, lax.rem(nchunk - 2, 2))
    wait_out(nchunk - 1, lax.rem(nchunk - 1, 2))


def _sample_body(idx_hbm, fx_hbm, fy_hbm, fz_hbm, tab_hbm,
                 r_hbm, g_hbm, b_hbm,
                 idx_v, f_v, rows_v, out_v, sems, *, bpw):
    wbase = _wid() * bpw
    nchunk = bpw // C
    lanes = lax.iota(jnp.int32, L)

    def stage_fire(ci, sl):
        base = wbase + ci * C
        pltpu.sync_copy(idx_hbm.at[pl.ds(base, C)], idx_v.at[sl])
        pltpu.sync_copy(fx_hbm.at[pl.ds(base, C)], f_v.at[sl, 0])
        pltpu.sync_copy(fy_hbm.at[pl.ds(base, C)], f_v.at[sl, 1])
        pltpu.sync_copy(fz_hbm.at[pl.ds(base, C)], f_v.at[sl, 2])
        pltpu.async_copy(tab_hbm.at[idx_v.at[sl]], rows_v.at[sl],
                         sems.at[sl])

    def wait_gathers(sl):
        pltpu.make_async_copy(tab_hbm.at[idx_v.at[sl]],
                              rows_v.at[sl], sems.at[sl]).wait()

    def blend_write(ci, sl):
        base = wbase + ci * C
        slv = jnp.full((L,), sl, jnp.int32)

        def blend_group(g, _):
            o = g * L
            rows = o + lanes
            fx = f_v[sl, 0, pl.ds(o, L)]
            fy = f_v[sl, 1, pl.ds(o, L)]
            fz = f_v[sl, 2, pl.ds(o, L)]
            gz = 1 - fz
            gy = 1 - fy
            gx = 1 - fx
            for c in range(3):

                def zlerp(dx, dy):
                    word = plsc.load_gather(
                        rows_v, [slv, rows,
                                 jnp.full((L,), dx * 6 + dy * 3 + c,
                                          jnp.int32)])
                    z0, z1 = plsc.unpack(
                        plsc.bitcast(word, jnp.bfloat16),
                        format=plsc.PackFormat.INTERLEAVED)
                    return z0 * gz + z1 * fz

                c0 = zlerp(0, 0) * gy + zlerp(0, 1) * fy
                c1 = zlerp(1, 0) * gy + zlerp(1, 1) * fy
                out_v[c, pl.ds(o, L)] = c0 * gx + c1 * fx
            return 0

        lax.fori_loop(0, C // L, blend_group, 0)
        pltpu.sync_copy(out_v.at[0], r_hbm.at[pl.ds(base, C)])
        pltpu.sync_copy(out_v.at[1], g_hbm.at[pl.ds(base, C)])
        pltpu.sync_copy(out_v.at[2], b_hbm.at[pl.ds(base, C)])

    stage_fire(0, 0)

    def pipe_step(ci, _):
        sl = lax.rem(ci, 2)
        nsl = lax.rem(ci + 1, 2)

        @pl.when(ci + 1 < nchunk)
        def _():
            stage_fire(ci + 1, nsl)

        wait_gathers(sl)
        blend_write(ci, sl)
        return 0

    lax.fori_loop(0, nchunk, pipe_step, 0)


def kernel(position, grid):
    b = position.shape[0]
    assert b % (NW * C) == 0
    bpw = b // NW
    mesh = plsc.VectorSubcoreMesh(core_axis_name="c", subcore_axis_name="s",
                                  num_cores=NC, num_subcores=NS)

    t = (position - VMIN) / (VMAX - VMIN) * (R - 1)
    t = jnp.clip(t, 0.0, R - 1 - 1e-6)
    i0 = jnp.floor(t).astype(jnp.int32)
    f = t - i0.astype(jnp.float32)
    cbase = (i0[:, 0] * R + i0[:, 1]) * R + i0[:, 2]
    fx, fy, fz = f[:, 0], f[:, 1], f[:, 2]

    gwords = jnp.concatenate(
        [jnp.transpose(grid, (0, 3, 1, 2)).reshape(-1),
         jnp.zeros((512,), jnp.float32)])

    table = pl.kernel(
        _build_body,
        out_type=jax.ShapeDtypeStruct((V, TW), jnp.float32),
        mesh=mesh,
        scratch_types=[
            pltpu.VMEM((2, 6, _SLAB), jnp.float32),
            pltpu.VMEM((2, CS, TW), jnp.float32),
            pltpu.SemaphoreType.DMA((2,)),
            pltpu.SemaphoreType.DMA((2,)),
        ],
        compiler_params=_SC_PARAMS,
    )(gwords)

    rgb = pl.kernel(
        functools.partial(_sample_body, bpw=bpw),
        out_type=[jax.ShapeDtypeStruct((b,), jnp.float32)] * 3,
        mesh=mesh,
        scratch_types=[
            pltpu.VMEM((2, C), jnp.int32),
            pltpu.VMEM((2, 3, C), jnp.float32),
            pltpu.VMEM((2, C, TW), jnp.float32),
            pltpu.VMEM((3, C), jnp.float32),
            pltpu.SemaphoreType.DMA((2,)),
        ],
        compiler_params=_SC_PARAMS,
    )(cbase, fx, fy, fz, table)

    return jnp.stack(rgb, axis=1)


# speedup vs baseline: 9.2196x; 1.0357x over previous
import functools

import jax
import jax.numpy as jnp
from jax import lax
from jax.experimental import pallas as pl
from jax.experimental.pallas import tpu as pltpu
from jax.experimental.pallas import tpu_sc as plsc

R = 128
VMIN = -3.0
VMAX = 3.0

NC, NS, L = 2, 16, 16
NW = NC * NS
V = R * R * R
TW = 16
CS = 2048
C = 2048

_SC_PARAMS = pltpu.CompilerParams(needs_layout_passes=False,
                                  use_tc_tiling_on_sc=False)


def _wid():
    return lax.axis_index("s") * NC + lax.axis_index("c")


_SLAB = CS + 136


def _build_body(src_hbm, tab_hbm, src_v, dst_v, sin, sout):
    rpw = V // NW
    base = _wid() * rpw
    nchunk = rpw // CS
    lanes = lax.iota(jnp.int32, L)

    def slab_start(a, dx, c):
        x = a // (R * R)
        yb = (a - x * R * R) // R
        xs = jnp.minimum(x + dx, R - 1)
        return pl.multiple_of(xs * (R * R * 3) + c * (R * R) + yb * R, 128)

    def stage(ci, sl):
        a = base + ci * CS
        for dx in (0, 1):
            for c in range(3):
                pltpu.async_copy(
                    src_hbm.at[pl.ds(slab_start(a, dx, c), _SLAB)],
                    src_v.at[sl, dx * 3 + c], sin.at[sl])

    def wait_stage(ci, sl):
        a = base + ci * CS
        for dx in (0, 1):
            for c in range(3):
                pltpu.make_async_copy(
                    src_hbm.at[pl.ds(slab_start(a, dx, c), _SLAB)],
                    src_v.at[sl, dx * 3 + c], sin.at[sl]).wait()

    def compute(ci, sl):
        a = base + ci * CS

        slv = jnp.full((L,), sl, jnp.int32)

        def yline(yl, _):
            for zg in range(8):
                zb = zg * L
                rows = (yl * 8 + zg) * L + lanes
                for dx in (0, 1):
                    for c in range(3):
                        s = dx * 3 + c
                        colw = jnp.full((L,), dx * 6 + c, jnp.int32)
                        for dy in (0, 1):
                            off = (yl + dy) * R + zb
                            v0 = src_v[sl, s, pl.ds(off, L)]
                            v1 = src_v[sl, s, pl.ds(off + 1, L)]
                            pair = plsc.bitcast(
                                plsc.pack(v0, v1,
                                          format=plsc.PackFormat.INTERLEAVED),
                                jnp.float32)
                            plsc.store_scatter(dst_v,
                                               [slv, rows, colw + 3 * dy],
                                               pair)
            return 0

        lax.fori_loop(0, CS // L // 8, yline, 0)
        pltpu.async_copy(dst_v.at[sl], tab_hbm.at[pl.ds(a, CS)], sout.at[sl])

    def wait_out(ci, sl):
        a = base + ci * CS
        pltpu.make_async_copy(dst_v.at[sl], tab_hbm.at[pl.ds(a, CS)],
                              sout.at[sl]).wait()

    stage(0, 0)

    def pipe_step(ci, _):
        sl = lax.rem(ci, 2)
        nsl = lax.rem(ci + 1, 2)

        @pl.when(ci + 1 < nchunk)
        def _():
            stage(ci + 1, nsl)

        wait_stage(ci, sl)

        @pl.when(ci >= 2)
        def _():
            wait_out(ci - 2, sl)

        compute(ci, sl)
        return 0

    lax.fori_loop(0, nchunk, pipe_step, 0)
    wait_out(nchunk - 2, lax.rem(nchunk - 2, 2))
    wait_out(nchunk - 1, lax.rem(nchunk - 1, 2))


def _sample_body(idx_hbm, fx_hbm, fy_hbm, fz_hbm, tab_hbm,
                 r_hbm, g_hbm, b_hbm,
                 idx_v, f_v, rows_v, out_v, sems, osems, *, bpw):
    wbase = _wid() * bpw
    nchunk = bpw // C
    lanes = lax.iota(jnp.int32, L)

    def stage_fire(ci, sl):
        base = wbase + ci * C
        pltpu.sync_copy(idx_hbm.at[pl.ds(base, C)], idx_v.at[sl])
        pltpu.sync_copy(fx_hbm.at[pl.ds(base, C)], f_v.at[sl, 0])
        pltpu.sync_copy(fy_hbm.at[pl.ds(base, C)], f_v.at[sl, 1])
        pltpu.sync_copy(fz_hbm.at[pl.ds(base, C)], f_v.at[sl, 2])
        pltpu.async_copy(tab_hbm.at[idx_v.at[sl]], rows_v.at[sl],
                         sems.at[sl])

    def wait_gathers(sl):
        pltpu.make_async_copy(tab_hbm.at[idx_v.at[sl]],
                              rows_v.at[sl], sems.at[sl]).wait()

    def blend_write(ci, sl):
        base = wbase + ci * C
        slv = jnp.full((L,), sl, jnp.int32)

        def blend_group(g, _):
            o = g * L
            rows = o + lanes
            fx = f_v[sl, 0, pl.ds(o, L)]
            fy = f_v[sl, 1, pl.ds(o, L)]
            fz = f_v[sl, 2, pl.ds(o, L)]
            gz = 1 - fz
            gy = 1 - fy
            gx = 1 - fx
            for c in range(3):

                def zlerp(dx, dy):
                    word = plsc.load_gather(
                        rows_v, [slv, rows,
                                 jnp.full((L,), dx * 6 + dy * 3 + c,
                                          jnp.int32)])
                    z0, z1 = plsc.unpack(
                        plsc.bitcast(word, jnp.bfloat16),
                        format=plsc.PackFormat.INTERLEAVED)
                    return z0 * gz + z1 * fz

                c0 = zlerp(0, 0) * gy + zlerp(0, 1) * fy
                c1 = zlerp(1, 0) * gy + zlerp(1, 1) * fy
                out_v[sl, c, pl.ds(o, L)] = c0 * gx + c1 * fx
            return 0

        lax.fori_loop(0, C // L, blend_group, 0)
        for ch, hbm in enumerate((r_hbm, g_hbm, b_hbm)):
            pltpu.async_copy(out_v.at[sl, ch], hbm.at[pl.ds(base, C)],
                             osems.at[sl])

    def wait_out(ci, sl):
        base = wbase + ci * C
        for ch, hbm in enumerate((r_hbm, g_hbm, b_hbm)):
            pltpu.make_async_copy(out_v.at[sl, ch], hbm.at[pl.ds(base, C)],
                                  osems.at[sl]).wait()

    stage_fire(0, 0)

    def pipe_step(ci, _):
        sl = lax.rem(ci, 2)
        nsl = lax.rem(ci + 1, 2)

        @pl.when(ci + 1 < nchunk)
        def _():
            stage_fire(ci + 1, nsl)

        wait_gathers(sl)

        @pl.when(ci >= 2)
        def _():
            wait_out(ci - 2, sl)

        blend_write(ci, sl)
        return 0

    lax.fori_loop(0, nchunk, pipe_step, 0)
    wait_out(nchunk - 2, lax.rem(nchunk - 2, 2))
    wait_out(nchunk - 1, lax.rem(nchunk - 1, 2))


def kernel(position, grid):
    b = position.shape[0]
    assert b % (NW * C) == 0
    bpw = b // NW
    mesh = plsc.VectorSubcoreMesh(core_axis_name="c", subcore_axis_name="s",
                                  num_cores=NC, num_subcores=NS)

    t = (position - VMIN) / (VMAX - VMIN) * (R - 1)
    t = jnp.clip(t, 0.0, R - 1 - 1e-6)
    i0 = jnp.floor(t).astype(jnp.int32)
    f = t - i0.astype(jnp.float32)
    cbase = (i0[:, 0] * R + i0[:, 1]) * R + i0[:, 2]
    fx, fy, fz = f[:, 0], f[:, 1], f[:, 2]

    gwords = jnp.concatenate(
        [jnp.transpose(grid, (0, 3, 1, 2)).reshape(-1),
         jnp.zeros((512,), jnp.float32)])

    table = pl.kernel(
        _build_body,
        out_type=jax.ShapeDtypeStruct((V, TW), jnp.float32),
        mesh=mesh,
        scratch_types=[
            pltpu.VMEM((2, 6, _SLAB), jnp.float32),
            pltpu.VMEM((2, CS, TW), jnp.float32),
            pltpu.SemaphoreType.DMA((2,)),
            pltpu.SemaphoreType.DMA((2,)),
        ],
        compiler_params=_SC_PARAMS,
    )(gwords)

    rgb = pl.kernel(
        functools.partial(_sample_body, bpw=bpw),
        out_type=[jax.ShapeDtypeStruct((b,), jnp.float32)] * 3,
        mesh=mesh,
        scratch_types=[
            pltpu.VMEM((2, C), jnp.int32),
            pltpu.VMEM((2, 3, C), jnp.float32),
            pltpu.VMEM((2, C, TW), jnp.float32),
            pltpu.VMEM((2, 3, C), jnp.float32),
            pltpu.SemaphoreType.DMA((2,)),
            pltpu.SemaphoreType.DMA((2,)),
        ],
        compiler_params=_SC_PARAMS,
    )(cbase, fx, fy, fz, table)

    return jnp.stack(rgb, axis=1)
